# Initial kernel scaffold; baseline (speedup 1.0000x reference)
import jax
import jax.numpy as jnp
from jax.experimental import pallas as pl


def kernel(coords, weight):
    raise NotImplementedError("write your pallas kernel here")


# speedup vs baseline: 1.0961x; 1.0961x over previous
import functools

import jax
import jax.numpy as jnp
from jax import lax
from jax.experimental import pallas as pl
from jax.experimental.pallas import tpu as pltpu
from jax.experimental.pallas import tpu_sc as plsc

NC, NS, L = 2, 16, 16
NW = NC * NS
CHUNK = 128
NBUF = 8
STEPS = CHUNK // L

_GRID = 512
_SCALE = jnp.float32(1e7)
_MAGIC = jnp.float32(8388608.0)


@functools.lru_cache(maxsize=None)
def _build(n_points, n_rows, feat):
    pts_per_w = n_points // NW
    nchunks = pts_per_w // CHUNK
    ngroups = nchunks // NBUF
    assert n_points == NW * CHUNK * NBUF * ngroups

    mesh = plsc.VectorSubcoreMesh(core_axis_name="c", subcore_axis_name="s")

    @functools.partial(
        pl.kernel,
        mesh=mesh,
        compiler_params=pltpu.CompilerParams(use_tc_tiling_on_sc=False),
        out_type=jax.ShapeDtypeStruct((n_points, feat), jnp.float32),
        scratch_types=(
            [pltpu.VMEM((pts_per_w,), jnp.float32),
             pltpu.VMEM((pts_per_w,), jnp.float32),
             pltpu.VMEM((NBUF, CHUNK), jnp.int32)]
            + [pltpu.VMEM((CHUNK, feat), jnp.float32) for _ in range(NBUF)]
            + [pltpu.SemaphoreType.DMA for _ in range(2 * NBUF)]
        ),
    )
    def qpool(xs_hbm, ys_hbm, weight_hbm, out_hbm, xs_v, ys_v, idx_v, *rest):
        bufs = rest[:NBUF]
        sem_g = rest[NBUF:2 * NBUF]
        sem_s = rest[2 * NBUF:3 * NBUF]

        wid = lax.axis_index("s") * NC + lax.axis_index("c")
        base = wid * pts_per_w

        pltpu.sync_copy(xs_hbm.at[pl.ds(base, pts_per_w)], xs_v)
        pltpu.sync_copy(ys_hbm.at[pl.ds(base, pts_per_w)], ys_v)

        def quantize(v):
            t = v * _SCALE
            r = jnp.where(t < _MAGIC, (t + _MAGIC) - _MAGIC, t)
            q = (r / _SCALE) * jnp.float32(_GRID)
            i = q.astype(jnp.int32)
            return jnp.minimum(jnp.maximum(i, 0), _GRID - 1)

        def compute_idx(chunk_j, slot):
            for s in range(STEPS):
                off = chunk_j * CHUNK + s * L
                x = xs_v[pl.ds(off, L)]
                y = ys_v[pl.ds(off, L)]
                idx_v[slot, pl.ds(s * L, L)] = quantize(x) * _GRID + quantize(y)

        def gather_start(slot):
            pltpu.async_copy(weight_hbm.at[idx_v.at[slot]], bufs[slot], sem_g[slot])

        def gather_wait(slot):
            pltpu.make_async_copy(
                weight_hbm.at[idx_v.at[slot]], bufs[slot], sem_g[slot]).wait()

        def scatter_start(slot, out_r0):
            pltpu.async_copy(bufs[slot], out_hbm.at[pl.ds(out_r0, CHUNK)],
                             sem_s[slot])

        def scatter_wait(slot):
            pltpu.make_async_copy(
                bufs[slot], out_hbm.at[pl.ds(0, CHUNK)], sem_s[slot]).wait()

        for b in range(NBUF):
            compute_idx(b, b)
            gather_start(b)

        def body(g, carry):
            for b in range(NBUF):
                j = g * NBUF + b
                gather_wait(b)
                scatter_start(b, base + j * CHUNK)
                compute_idx(j + NBUF, b)
                scatter_wait(b)
                gather_start(b)
            return carry

        lax.fori_loop(0, ngroups - 1, body, 0)

        for b in range(NBUF):
            j = (ngroups - 1) * NBUF + b
            gather_wait(b)
            scatter_start(b, base + j * CHUNK)
        for b in range(NBUF):
            scatter_wait(b)

    return qpool


def kernel(coords, weight):
    n_points = coords.shape[0]
    n_rows, feat = weight.shape
    xs = coords[:, 0]
    ys = coords[:, 1]
    return _build(n_points, n_rows, feat)(xs, ys, weight)


# speedup vs baseline: 1.1008x; 1.0044x over previous
import functools

import jax
import jax.numpy as jnp
from jax import lax
from jax.experimental import pallas as pl
from jax.experimental.pallas import tpu as pltpu
from jax.experimental.pallas import tpu_sc as plsc

NC, NS, L = 2, 16, 16
NW = NC * NS
CHUNK = 128
NBUF = 8
STEPS = CHUNK // L

_GRID = 512
_SCALE = 1e7
_MAGIC = 8388608.0
_Q = float(jnp.float32(512.0) / jnp.float32(1e7))


@functools.lru_cache(maxsize=None)
def _build(n_points, n_rows, feat):
    pts_per_w = n_points // NW
    nchunks = pts_per_w // CHUNK
    ngroups = nchunks // NBUF
    assert n_points == NW * CHUNK * NBUF * ngroups

    mesh = plsc.VectorSubcoreMesh(core_axis_name="c", subcore_axis_name="s")

    @functools.partial(
        pl.kernel,
        mesh=mesh,
        compiler_params=pltpu.CompilerParams(use_tc_tiling_on_sc=False),
        out_type=jax.ShapeDtypeStruct((n_points, feat), jnp.float32),
        scratch_types=(
            [pltpu.VMEM((pts_per_w,), jnp.float32),
             pltpu.VMEM((pts_per_w,), jnp.float32),
             pltpu.VMEM((NBUF, CHUNK), jnp.int32)]
            + [pltpu.VMEM((CHUNK, feat), jnp.float32) for _ in range(NBUF)]
            + [pltpu.SemaphoreType.DMA for _ in range(2 * NBUF)]
        ),
    )
    def qpool(xs_hbm, ys_hbm, weight_hbm, out_hbm, xs_v, ys_v, idx_v, *rest):
        bufs = rest[:NBUF]
        sem_g = rest[NBUF:2 * NBUF]
        sem_s = rest[2 * NBUF:3 * NBUF]

        wid = lax.axis_index("s") * NC + lax.axis_index("c")
        base = wid * pts_per_w

        pltpu.sync_copy(xs_hbm.at[pl.ds(base, pts_per_w)], xs_v)
        pltpu.sync_copy(ys_hbm.at[pl.ds(base, pts_per_w)], ys_v)

        def quantize(v):
            t = v * jnp.float32(_SCALE)
            r = jnp.where(t < jnp.float32(_MAGIC),
                          (t + jnp.float32(_MAGIC)) - jnp.float32(_MAGIC),
                          t)
            i = (r * jnp.float32(_Q)).astype(jnp.int32)
            return jnp.minimum(i, _GRID - 1)

        def compute_idx(chunk_j, slot):
            for s in range(STEPS):
                off = chunk_j * CHUNK + s * L
                x = xs_v[pl.ds(off, L)]
                y = ys_v[pl.ds(off, L)]
                idx_v[slot, pl.ds(s * L, L)] = quantize(x) * _GRID + quantize(y)

        def gather_start(slot):
            pltpu.async_copy(weight_hbm.at[idx_v.at[slot]], bufs[slot],
                             sem_g[slot])

        def gather_wait(slot):
            pltpu.make_async_copy(
                weight_hbm.at[idx_v.at[slot]], bufs[slot], sem_g[slot]).wait()

        def scatter_start(slot, out_r0):
            pltpu.async_copy(bufs[slot], out_hbm.at[pl.ds(out_r0, CHUNK)],
                             sem_s[slot])

        def scatter_wait(slot):
            pltpu.make_async_copy(
                bufs[slot], out_hbm.at[pl.ds(0, CHUNK)], sem_s[slot]).wait()

        for b in range(NBUF):
            compute_idx(b, b)
            gather_start(b)

        def body(g, carry):
            for b in range(NBUF):
                j = g * NBUF + b
                gather_wait(b)
                scatter_start(b, base + j * CHUNK)
                compute_idx(j + NBUF, b)
                scatter_wait(b)
                gather_start(b)
            return carry

        lax.fori_loop(0, ngroups - 1, body, 0)

        for b in range(NBUF):
            j = (ngroups - 1) * NBUF + b
            gather_wait(b)
            scatter_start(b, base + j * CHUNK)
        for b in range(NBUF):
            scatter_wait(b)

    return qpool


def kernel(coords, weight):
    n_points = coords.shape[0]
    n_rows, feat = weight.shape
    xs = coords[:, 0]
    ys = coords[:, 1]
    return _build(n_points, n_rows, feat)(xs, ys, weight)
